# speedup vs baseline: 9.7505x; 9.7505x over previous
import functools

import jax
import jax.numpy as jnp
from jax import lax
from jax.experimental import pallas as pl
from jax.experimental.pallas import tpu as pltpu
from jax.experimental.pallas import tpu_sc as plsc

B = 8
T = 8192
D = 256
MAX_LEN = 2048
BINS = 80
CHUNK = 256
NCHUNKS = MAX_LEN // CHUNK
NWORKERS = 32
ROWS_PER_W = T // NWORKERS
IDX_W = 128


def _sc_scatter(flat, idx):
    mesh = plsc.VectorSubcoreMesh(core_axis_name="c", subcore_axis_name="s")

    @functools.partial(
        pl.kernel,
        out_type=jax.ShapeDtypeStruct((MAX_LEN * B, D), jnp.float32),
        mesh=mesh,
        scratch_types=[
            pltpu.VMEM((2, IDX_W), jnp.int32),
            pltpu.VMEM((ROWS_PER_W, D), jnp.float32),
            pltpu.SemaphoreType.DMA,
        ],
    )
    def scatter_kernel(flat_hbm, idx_hbm, out_hbm, idx_v, rows_v, sem):
        wid = lax.axis_index("s") * 2 + lax.axis_index("c")
        base = wid * ROWS_PER_W
        pltpu.sync_copy(idx_hbm.at[wid], idx_v)
        pltpu.sync_copy(flat_hbm.at[pl.ds(base, ROWS_PER_W)], rows_v)
        cp0 = pltpu.async_copy(rows_v.at[pl.ds(0, IDX_W)],
                               out_hbm.at[idx_v.at[0]], sem)
        cp1 = pltpu.async_copy(rows_v.at[pl.ds(IDX_W, IDX_W)],
                               out_hbm.at[idx_v.at[1]], sem)
        cp0.wait()
        cp1.wait()

    return scatter_kernel(flat, idx)


def _tc_body(xf_ref, xb_ref, lens_ref, wif_ref, whf_ref, bcf_ref, bhnf_ref,
             wib_ref, whb_ref, bcb_ref, bhnb_ref, w1_ref, w2_ref, w3s_ref,
             w3e_ref, w3t_ref, sw_ref, ew_ref, out_ref,
             gif_ref, gib_ref, hf_ref, hb_ref):
    k = pl.program_id(0)

    @pl.when(k == 0)
    def _init():
        hf_ref[...] = jnp.zeros((B, D), jnp.float32)
        hb_ref[...] = jnp.zeros((B, D), jnp.float32)

    gif_ref[...] = (
        jnp.dot(xf_ref[...], wif_ref[...], preferred_element_type=jnp.float32)
        + bcf_ref[...])
    gib_ref[...] = (
        jnp.dot(xb_ref[...], wib_ref[...], preferred_element_type=jnp.float32)
        + bcb_ref[...])

    lens = lens_ref[...]
    base_t = k * CHUNK

    def step(j, carry):
        hf = hf_ref[...]
        hb = hb_ref[...]
        rf = pl.multiple_of(j * B, B)
        rb = pl.multiple_of((CHUNK - 1 - j) * B, B)
        gif = gif_ref[pl.ds(rf, B), :]
        gib = gib_ref[pl.ds(rb, B), :]
        ghf = jnp.dot(hf, whf_ref[...], preferred_element_type=jnp.float32)
        ghb = jnp.dot(hb, whb_ref[...], preferred_element_type=jnp.float32)

        tf = base_t + j
        rzf = jax.nn.sigmoid(gif[:, :2 * D] + ghf[:, :2 * D])
        nf = jnp.tanh(gif[:, 2 * D:]
                      + rzf[:, :D] * (ghf[:, 2 * D:] + bhnf_ref[...]))
        hf_new = (1.0 - rzf[:, D:]) * nf + rzf[:, D:] * hf
        hf_ref[...] = jnp.where(lens > tf, hf_new, hf)

        tb = (MAX_LEN - 1) - tf
        rzb = jax.nn.sigmoid(gib[:, :2 * D] + ghb[:, :2 * D])
        nb = jnp.tanh(gib[:, 2 * D:]
                      + rzb[:, :D] * (ghb[:, 2 * D:] + bhnb_ref[...]))
        hb_new = (1.0 - rzb[:, D:]) * nb + rzb[:, D:] * hb
        hb_ref[...] = jnp.where(lens > tb, hb_new, hb)
        return carry

    lax.fori_loop(0, CHUNK, step, 0)

    @pl.when(k == NCHUNKS - 1)
    def _head():
        h = jnp.concatenate([hf_ref[...], hb_ref[...]], axis=1)
        u1 = jnp.maximum(
            jnp.dot(h, w1_ref[...], preferred_element_type=jnp.float32), 0.0)
        u2 = jnp.maximum(
            jnp.dot(u1, w2_ref[...], preferred_element_type=jnp.float32), 0.0)
        outs = jnp.dot(u2, w3s_ref[...], preferred_element_type=jnp.float32)
        oute = jnp.dot(u2, w3e_ref[...], preferred_element_type=jnp.float32)
        outt = jnp.dot(u2, w3t_ref[...], preferred_element_type=jnp.float32)

        def soft_off(o, w):
            m = jnp.max(o, axis=1, keepdims=True)
            e = jnp.exp(o - m)
            return (jnp.sum(e * w, axis=1, keepdims=True)
                    / jnp.sum(e, axis=1, keepdims=True))

        so = soft_off(outs, sw_ref[...])
        eo = soft_off(oute, ew_ref[...])
        out_ref[...] = jnp.concatenate([so, eo, outt], axis=1)


def _tc_gru(padded, lens, Wi_f, Wh_f, bc_f, bhn_f, Wi_b, Wh_b, bc_b, bhn_b,
            W1, W2, W3s, W3e, W3t, sw, ew):
    const = lambda shape: pl.BlockSpec(shape, lambda k: (0, 0))
    return pl.pallas_call(
        _tc_body,
        grid=(NCHUNKS,),
        in_specs=[
            pl.BlockSpec((CHUNK * B, D), lambda k: (k, 0)),
            pl.BlockSpec((CHUNK * B, D), lambda k: (NCHUNKS - 1 - k, 0)),
            const((B, 1)),
            const((D, 3 * D)), const((D, 3 * D)),
            const((1, 3 * D)), const((1, D)),
            const((D, 3 * D)), const((D, 3 * D)),
            const((1, 3 * D)), const((1, D)),
            const((2 * D, 256)), const((256, 256)),
            const((256, BINS)), const((256, BINS)), const((256, 5)),
            const((1, BINS)), const((1, BINS)),
        ],
        out_specs=pl.BlockSpec((B, 7), lambda k: (0, 0)),
        out_shape=jax.ShapeDtypeStruct((B, 7), jnp.float32),
        scratch_shapes=[
            pltpu.VMEM((CHUNK * B, 3 * D), jnp.float32),
            pltpu.VMEM((CHUNK * B, 3 * D), jnp.float32),
            pltpu.VMEM((B, D), jnp.float32),
            pltpu.VMEM((B, D), jnp.float32),
        ],
    )(padded, padded, lens, Wi_f, Wh_f, bc_f, bhn_f, Wi_b, Wh_b, bc_b,
      bhn_b, W1, W2, W3s, W3e, W3t, sw, ew)


def kernel(flat, cu_seqlens, Wi_f, Wh_f, bi_f, bh_f, Wi_b, Wh_b, bi_b, bh_b,
           W1, W2, W3, start_w, end_w):
    cu = cu_seqlens.astype(jnp.int32)
    tok = jnp.arange(T, dtype=jnp.int32)
    seg = jnp.searchsorted(cu, tok, side="right").astype(jnp.int32) - 1
    pos = tok - cu[seg]
    dest = (pos * B + seg).reshape(NWORKERS, 2, IDX_W)
    padded = _sc_scatter(flat, dest)

    lens = (cu[1:] - cu[:-1]).reshape(B, 1)
    zero_n = jnp.zeros((D,), jnp.float32)
    bc_f = (bi_f + jnp.concatenate([bh_f[:2 * D], zero_n])).reshape(1, 3 * D)
    bc_b = (bi_b + jnp.concatenate([bh_b[:2 * D], zero_n])).reshape(1, 3 * D)
    bhn_f = bh_f[2 * D:].reshape(1, D)
    bhn_b = bh_b[2 * D:].reshape(1, D)
    W3s = W3[:, :BINS]
    W3e = W3[:, BINS:2 * BINS]
    W3t = W3[:, 2 * BINS:]
    sw = start_w.reshape(1, BINS)
    ew = end_w.reshape(1, BINS)

    return _tc_gru(padded, lens, Wi_f, Wh_f, bc_f, bhn_f, Wi_b, Wh_b,
                   bc_b, bhn_b, W1, W2, W3s, W3e, W3t, sw, ew)


# speedup vs baseline: 9.9237x; 1.0178x over previous
import functools

import jax
import jax.numpy as jnp
from jax import lax
from jax.experimental import pallas as pl
from jax.experimental.pallas import tpu as pltpu
from jax.experimental.pallas import tpu_sc as plsc

B = 8
T = 8192
D = 256
MAX_LEN = 2048
BINS = 80
CHUNK = 256
NCHUNKS = MAX_LEN // CHUNK
NWORKERS = 32
ROWS_PER_W = T // NWORKERS
IDX_W = 128


def _sc_scatter(flat, idx):
    mesh = plsc.VectorSubcoreMesh(core_axis_name="c", subcore_axis_name="s")

    @functools.partial(
        pl.kernel,
        out_type=jax.ShapeDtypeStruct((MAX_LEN * B, D), jnp.float32),
        mesh=mesh,
        scratch_types=[
            pltpu.VMEM((2, IDX_W), jnp.int32),
            pltpu.VMEM((ROWS_PER_W, D), jnp.float32),
            pltpu.SemaphoreType.DMA,
        ],
    )
    def scatter_kernel(flat_hbm, idx_hbm, out_hbm, idx_v, rows_v, sem):
        wid = lax.axis_index("s") * 2 + lax.axis_index("c")
        base = wid * ROWS_PER_W
        pltpu.sync_copy(idx_hbm.at[wid], idx_v)
        pltpu.sync_copy(flat_hbm.at[pl.ds(base, ROWS_PER_W)], rows_v)
        cp0 = pltpu.async_copy(rows_v.at[pl.ds(0, IDX_W)],
                               out_hbm.at[idx_v.at[0]], sem)
        cp1 = pltpu.async_copy(rows_v.at[pl.ds(IDX_W, IDX_W)],
                               out_hbm.at[idx_v.at[1]], sem)
        cp0.wait()
        cp1.wait()

    return scatter_kernel(flat, idx)


def _tc_body(xf_ref, xb_ref, lens_ref, wif_ref, whf_ref, bcf_ref, bhnf_ref,
             wib_ref, whb_ref, bcb_ref, bhnb_ref, w1_ref, w2_ref, w3s_ref,
             w3e_ref, w3t_ref, sw_ref, ew_ref, out_ref,
             gif_ref, gib_ref, hf_ref, hb_ref):
    k = pl.program_id(0)

    @pl.when(k == 0)
    def _init():
        hf_ref[...] = jnp.zeros((B, D), jnp.float32)
        hb_ref[...] = jnp.zeros((B, D), jnp.float32)

    gif_ref[...] = (
        jnp.dot(xf_ref[...].astype(jnp.bfloat16), wif_ref[...],
                preferred_element_type=jnp.float32)
        + bcf_ref[...])
    gib_ref[...] = (
        jnp.dot(xb_ref[...].astype(jnp.bfloat16), wib_ref[...],
                preferred_element_type=jnp.float32)
        + bcb_ref[...])

    lens = lens_ref[...]
    base_t = k * CHUNK

    def step(j, carry):
        hf = hf_ref[...]
        hb = hb_ref[...]
        rf = pl.multiple_of(j * B, B)
        rb = pl.multiple_of((CHUNK - 1 - j) * B, B)
        gif = gif_ref[pl.ds(rf, B), :]
        gib = gib_ref[pl.ds(rb, B), :]
        ghf = jnp.dot(hf.astype(jnp.bfloat16), whf_ref[...],
                      preferred_element_type=jnp.float32)
        ghb = jnp.dot(hb.astype(jnp.bfloat16), whb_ref[...],
                      preferred_element_type=jnp.float32)

        tf = base_t + j
        rzf = jax.nn.sigmoid(gif[:, :2 * D] + ghf[:, :2 * D])
        nf = jnp.tanh(gif[:, 2 * D:]
                      + rzf[:, :D] * (ghf[:, 2 * D:] + bhnf_ref[...]))
        hf_new = (1.0 - rzf[:, D:]) * nf + rzf[:, D:] * hf
        hf_ref[...] = jnp.where(lens > tf, hf_new, hf)

        tb = (MAX_LEN - 1) - tf
        rzb = jax.nn.sigmoid(gib[:, :2 * D] + ghb[:, :2 * D])
        nb = jnp.tanh(gib[:, 2 * D:]
                      + rzb[:, :D] * (ghb[:, 2 * D:] + bhnb_ref[...]))
        hb_new = (1.0 - rzb[:, D:]) * nb + rzb[:, D:] * hb
        hb_ref[...] = jnp.where(lens > tb, hb_new, hb)
        return carry

    lax.fori_loop(0, CHUNK, step, 0)

    @pl.when(k == NCHUNKS - 1)
    def _head():
        h = jnp.concatenate([hf_ref[...], hb_ref[...]], axis=1)
        u1 = jnp.maximum(
            jnp.dot(h, w1_ref[...], preferred_element_type=jnp.float32), 0.0)
        u2 = jnp.maximum(
            jnp.dot(u1, w2_ref[...], preferred_element_type=jnp.float32), 0.0)
        outs = jnp.dot(u2, w3s_ref[...], preferred_element_type=jnp.float32)
        oute = jnp.dot(u2, w3e_ref[...], preferred_element_type=jnp.float32)
        outt = jnp.dot(u2, w3t_ref[...], preferred_element_type=jnp.float32)

        def soft_off(o, w):
            m = jnp.max(o, axis=1, keepdims=True)
            e = jnp.exp(o - m)
            return (jnp.sum(e * w, axis=1, keepdims=True)
                    / jnp.sum(e, axis=1, keepdims=True))

        so = soft_off(outs, sw_ref[...])
        eo = soft_off(oute, ew_ref[...])
        out_ref[...] = jnp.concatenate([so, eo, outt], axis=1)


def _tc_gru(padded, lens, Wi_f, Wh_f, bc_f, bhn_f, Wi_b, Wh_b, bc_b, bhn_b,
            W1, W2, W3s, W3e, W3t, sw, ew):
    const = lambda shape: pl.BlockSpec(shape, lambda k: (0, 0))
    return pl.pallas_call(
        _tc_body,
        grid=(NCHUNKS,),
        in_specs=[
            pl.BlockSpec((CHUNK * B, D), lambda k: (k, 0)),
            pl.BlockSpec((CHUNK * B, D), lambda k: (NCHUNKS - 1 - k, 0)),
            const((B, 1)),
            const((D, 3 * D)), const((D, 3 * D)),
            const((1, 3 * D)), const((1, D)),
            const((D, 3 * D)), const((D, 3 * D)),
            const((1, 3 * D)), const((1, D)),
            const((2 * D, 256)), const((256, 256)),
            const((256, BINS)), const((256, BINS)), const((256, 5)),
            const((1, BINS)), const((1, BINS)),
        ],
        out_specs=pl.BlockSpec((B, 7), lambda k: (0, 0)),
        out_shape=jax.ShapeDtypeStruct((B, 7), jnp.float32),
        scratch_shapes=[
            pltpu.VMEM((CHUNK * B, 3 * D), jnp.float32),
            pltpu.VMEM((CHUNK * B, 3 * D), jnp.float32),
            pltpu.VMEM((B, D), jnp.float32),
            pltpu.VMEM((B, D), jnp.float32),
        ],
    )(padded, padded, lens, Wi_f, Wh_f, bc_f, bhn_f, Wi_b, Wh_b, bc_b,
      bhn_b, W1, W2, W3s, W3e, W3t, sw, ew)


def kernel(flat, cu_seqlens, Wi_f, Wh_f, bi_f, bh_f, Wi_b, Wh_b, bi_b, bh_b,
           W1, W2, W3, start_w, end_w):
    cu = cu_seqlens.astype(jnp.int32)
    tok = jnp.arange(T, dtype=jnp.int32)
    seg = jnp.searchsorted(cu, tok, side="right").astype(jnp.int32) - 1
    pos = tok - cu[seg]
    dest = (pos * B + seg).reshape(NWORKERS, 2, IDX_W)
    padded = _sc_scatter(flat, dest)

    lens = (cu[1:] - cu[:-1]).reshape(B, 1)
    zero_n = jnp.zeros((D,), jnp.float32)
    bc_f = (bi_f + jnp.concatenate([bh_f[:2 * D], zero_n])).reshape(1, 3 * D)
    bc_b = (bi_b + jnp.concatenate([bh_b[:2 * D], zero_n])).reshape(1, 3 * D)
    bhn_f = bh_f[2 * D:].reshape(1, D)
    bhn_b = bh_b[2 * D:].reshape(1, D)
    W3s = W3[:, :BINS]
    W3e = W3[:, BINS:2 * BINS]
    W3t = W3[:, 2 * BINS:]
    sw = start_w.reshape(1, BINS)
    ew = end_w.reshape(1, BINS)

    bf16 = jnp.bfloat16
    return _tc_gru(padded, lens, Wi_f.astype(bf16), Wh_f.astype(bf16),
                   bc_f, bhn_f, Wi_b.astype(bf16), Wh_b.astype(bf16),
                   bc_b, bhn_b, W1, W2, W3s, W3e, W3t, sw, ew)


# speedup vs baseline: 12.7383x; 1.2836x over previous
import functools

import jax
import jax.numpy as jnp
from jax import lax
from jax.experimental import pallas as pl
from jax.experimental.pallas import tpu as pltpu
from jax.experimental.pallas import tpu_sc as plsc

B = 8
T = 8192
D = 256
MAX_LEN = 2048
BINS = 80
CHUNK = 256
NCHUNKS = MAX_LEN // CHUNK
NWORKERS = 32
ROWS_PER_W = T // NWORKERS
IDX_W = 128


def _sc_scatter(flat, idx):
    mesh = plsc.VectorSubcoreMesh(core_axis_name="c", subcore_axis_name="s")

    @functools.partial(
        pl.kernel,
        out_type=jax.ShapeDtypeStruct((MAX_LEN * B, D), jnp.float32),
        mesh=mesh,
        scratch_types=[
            pltpu.VMEM((2, IDX_W), jnp.int32),
            pltpu.VMEM((ROWS_PER_W, D), jnp.float32),
            pltpu.SemaphoreType.DMA,
        ],
    )
    def scatter_kernel(flat_hbm, idx_hbm, out_hbm, idx_v, rows_v, sem):
        wid = lax.axis_index("s") * 2 + lax.axis_index("c")
        base = wid * ROWS_PER_W
        pltpu.sync_copy(idx_hbm.at[wid], idx_v)
        pltpu.sync_copy(flat_hbm.at[pl.ds(base, ROWS_PER_W)], rows_v)
        cp0 = pltpu.async_copy(rows_v.at[pl.ds(0, IDX_W)],
                               out_hbm.at[idx_v.at[0]], sem)
        cp1 = pltpu.async_copy(rows_v.at[pl.ds(IDX_W, IDX_W)],
                               out_hbm.at[idx_v.at[1]], sem)
        cp0.wait()
        cp1.wait()

    return scatter_kernel(flat, idx)


def _tc_body(xf_ref, xb_ref, lens_ref, wif_ref, whf_ref, bcf_ref, bhnf_ref,
             wib_ref, whb_ref, bcb_ref, bhnb_ref, w1_ref, w2_ref, w3s_ref,
             w3e_ref, w3t_ref, sw_ref, ew_ref, out_ref,
             gif_ref, gib_ref, hf_ref, hb_ref):
    k = pl.program_id(0)

    @pl.when(k == 0)
    def _init():
        hf_ref[...] = jnp.zeros((B, D), jnp.float32)
        hb_ref[...] = jnp.zeros((B, D), jnp.float32)

    gif_ref[...] = (
        jnp.dot(xf_ref[...].astype(jnp.bfloat16), wif_ref[...],
                preferred_element_type=jnp.float32)
        + bcf_ref[...])
    gib_ref[...] = (
        jnp.dot(xb_ref[...].astype(jnp.bfloat16), wib_ref[...],
                preferred_element_type=jnp.float32)
        + bcb_ref[...])

    lens = lens_ref[...]
    base_t = k * CHUNK

    def step(j, carry):
        hf, hb = carry
        rf = pl.multiple_of(j * B, B)
        rb = pl.multiple_of((CHUNK - 1 - j) * B, B)
        gif = gif_ref[pl.ds(rf, B), :]
        gib = gib_ref[pl.ds(rb, B), :]
        ghf = jnp.dot(hf.astype(jnp.bfloat16), whf_ref[...],
                      preferred_element_type=jnp.float32)
        ghb = jnp.dot(hb.astype(jnp.bfloat16), whb_ref[...],
                      preferred_element_type=jnp.float32)

        tf = base_t + j
        rzf = jax.nn.sigmoid(gif[:, :2 * D] + ghf[:, :2 * D])
        nf = jnp.tanh(gif[:, 2 * D:]
                      + rzf[:, :D] * (ghf[:, 2 * D:] + bhnf_ref[...]))
        hf_new = nf + rzf[:, D:] * (hf - nf)
        hf = jnp.where(lens > tf, hf_new, hf)

        tb = (MAX_LEN - 1) - tf
        rzb = jax.nn.sigmoid(gib[:, :2 * D] + ghb[:, :2 * D])
        nb = jnp.tanh(gib[:, 2 * D:]
                      + rzb[:, :D] * (ghb[:, 2 * D:] + bhnb_ref[...]))
        hb_new = nb + rzb[:, D:] * (hb - nb)
        hb = jnp.where(lens > tb, hb_new, hb)
        return hf, hb

    hf_fin, hb_fin = lax.fori_loop(0, CHUNK, step,
                                   (hf_ref[...], hb_ref[...]), unroll=8)
    hf_ref[...] = hf_fin
    hb_ref[...] = hb_fin

    @pl.when(k == NCHUNKS - 1)
    def _head():
        h = jnp.concatenate([hf_ref[...], hb_ref[...]], axis=1)
        u1 = jnp.maximum(
            jnp.dot(h, w1_ref[...], preferred_element_type=jnp.float32), 0.0)
        u2 = jnp.maximum(
            jnp.dot(u1, w2_ref[...], preferred_element_type=jnp.float32), 0.0)
        outs = jnp.dot(u2, w3s_ref[...], preferred_element_type=jnp.float32)
        oute = jnp.dot(u2, w3e_ref[...], preferred_element_type=jnp.float32)
        outt = jnp.dot(u2, w3t_ref[...], preferred_element_type=jnp.float32)

        def soft_off(o, w):
            m = jnp.max(o, axis=1, keepdims=True)
            e = jnp.exp(o - m)
            return (jnp.sum(e * w, axis=1, keepdims=True)
                    / jnp.sum(e, axis=1, keepdims=True))

        so = soft_off(outs, sw_ref[...])
        eo = soft_off(oute, ew_ref[...])
        out_ref[...] = jnp.concatenate([so, eo, outt], axis=1)


def _tc_gru(padded, lens, Wi_f, Wh_f, bc_f, bhn_f, Wi_b, Wh_b, bc_b, bhn_b,
            W1, W2, W3s, W3e, W3t, sw, ew):
    const = lambda shape: pl.BlockSpec(shape, lambda k: (0, 0))
    return pl.pallas_call(
        _tc_body,
        grid=(NCHUNKS,),
        in_specs=[
            pl.BlockSpec((CHUNK * B, D), lambda k: (k, 0)),
            pl.BlockSpec((CHUNK * B, D), lambda k: (NCHUNKS - 1 - k, 0)),
            const((B, 1)),
            const((D, 3 * D)), const((D, 3 * D)),
            const((1, 3 * D)), const((1, D)),
            const((D, 3 * D)), const((D, 3 * D)),
            const((1, 3 * D)), const((1, D)),
            const((2 * D, 256)), const((256, 256)),
            const((256, BINS)), const((256, BINS)), const((256, 5)),
            const((1, BINS)), const((1, BINS)),
        ],
        out_specs=pl.BlockSpec((B, 7), lambda k: (0, 0)),
        out_shape=jax.ShapeDtypeStruct((B, 7), jnp.float32),
        scratch_shapes=[
            pltpu.VMEM((CHUNK * B, 3 * D), jnp.float32),
            pltpu.VMEM((CHUNK * B, 3 * D), jnp.float32),
            pltpu.VMEM((B, D), jnp.float32),
            pltpu.VMEM((B, D), jnp.float32),
        ],
    )(padded, padded, lens, Wi_f, Wh_f, bc_f, bhn_f, Wi_b, Wh_b, bc_b,
      bhn_b, W1, W2, W3s, W3e, W3t, sw, ew)


def kernel(flat, cu_seqlens, Wi_f, Wh_f, bi_f, bh_f, Wi_b, Wh_b, bi_b, bh_b,
           W1, W2, W3, start_w, end_w):
    cu = cu_seqlens.astype(jnp.int32)
    tok = jnp.arange(T, dtype=jnp.int32)
    seg = jnp.searchsorted(cu, tok, side="right").astype(jnp.int32) - 1
    pos = tok - cu[seg]
    dest = (pos * B + seg).reshape(NWORKERS, 2, IDX_W)
    padded = _sc_scatter(flat, dest)

    lens = (cu[1:] - cu[:-1]).reshape(B, 1)
    zero_n = jnp.zeros((D,), jnp.float32)
    bc_f = (bi_f + jnp.concatenate([bh_f[:2 * D], zero_n])).reshape(1, 3 * D)
    bc_b = (bi_b + jnp.concatenate([bh_b[:2 * D], zero_n])).reshape(1, 3 * D)
    bhn_f = bh_f[2 * D:].reshape(1, D)
    bhn_b = bh_b[2 * D:].reshape(1, D)
    W3s = W3[:, :BINS]
    W3e = W3[:, BINS:2 * BINS]
    W3t = W3[:, 2 * BINS:]
    sw = start_w.reshape(1, BINS)
    ew = end_w.reshape(1, BINS)

    bf16 = jnp.bfloat16
    return _tc_gru(padded, lens, Wi_f.astype(bf16), Wh_f.astype(bf16),
                   bc_f, bhn_f, Wi_b.astype(bf16), Wh_b.astype(bf16),
                   bc_b, bhn_b, W1, W2, W3s, W3e, W3t, sw, ew)


# speedup vs baseline: 12.8050x; 1.0052x over previous
import functools

import jax
import jax.numpy as jnp
from jax import lax
from jax.experimental import pallas as pl
from jax.experimental.pallas import tpu as pltpu
from jax.experimental.pallas import tpu_sc as plsc

B = 8
T = 8192
D = 256
MAX_LEN = 2048
BINS = 80
CHUNK = 256
NCHUNKS = MAX_LEN // CHUNK
NWORKERS = 32
ROWS_PER_W = T // NWORKERS
IDX_W = 128


def _sc_scatter(flat, idx):
    mesh = plsc.VectorSubcoreMesh(core_axis_name="c", subcore_axis_name="s")

    @functools.partial(
        pl.kernel,
        out_type=jax.ShapeDtypeStruct((MAX_LEN * B, D), jnp.float32),
        mesh=mesh,
        scratch_types=[
            pltpu.VMEM((2, IDX_W), jnp.int32),
            pltpu.VMEM((ROWS_PER_W, D), jnp.float32),
            pltpu.SemaphoreType.DMA,
            pltpu.SemaphoreType.DMA,
            pltpu.SemaphoreType.DMA,
        ],
    )
    def scatter_kernel(flat_hbm, idx_hbm, out_hbm, idx_v, rows_v,
                       sem_ld0, sem_ld1, sem_st):
        wid = lax.axis_index("s") * 2 + lax.axis_index("c")
        base = wid * ROWS_PER_W
        ld0 = pltpu.async_copy(flat_hbm.at[pl.ds(base, IDX_W)],
                               rows_v.at[pl.ds(0, IDX_W)], sem_ld0)
        ld1 = pltpu.async_copy(flat_hbm.at[pl.ds(base + IDX_W, IDX_W)],
                               rows_v.at[pl.ds(IDX_W, IDX_W)], sem_ld1)
        pltpu.sync_copy(idx_hbm.at[wid], idx_v)
        ld0.wait()
        st0 = pltpu.async_copy(rows_v.at[pl.ds(0, IDX_W)],
                               out_hbm.at[idx_v.at[0]], sem_st)
        ld1.wait()
        st1 = pltpu.async_copy(rows_v.at[pl.ds(IDX_W, IDX_W)],
                               out_hbm.at[idx_v.at[1]], sem_st)
        st0.wait()
        st1.wait()

    return scatter_kernel(flat, idx)


def _tc_body(xf_ref, xb_ref, lens_ref, wif_ref, whf_ref, bcf_ref, bhnf_ref,
             wib_ref, whb_ref, bcb_ref, bhnb_ref, w1_ref, w2_ref, w3s_ref,
             w3e_ref, w3t_ref, sw_ref, ew_ref, out_ref,
             gif_ref, gib_ref, hf_ref, hb_ref):
    k = pl.program_id(0)

    @pl.when(k == 0)
    def _init():
        hf_ref[...] = jnp.zeros((B, D), jnp.float32)
        hb_ref[...] = jnp.zeros((B, D), jnp.float32)

    gif_ref[...] = (
        jnp.dot(xf_ref[...].astype(jnp.bfloat16), wif_ref[...],
                preferred_element_type=jnp.float32)
        + bcf_ref[...])
    gib_ref[...] = (
        jnp.dot(xb_ref[...].astype(jnp.bfloat16), wib_ref[...],
                preferred_element_type=jnp.float32)
        + bcb_ref[...])

    lens = lens_ref[...]
    base_t = k * CHUNK

    def step(j, carry):
        hf, hb = carry
        rf = pl.multiple_of(j * B, B)
        rb = pl.multiple_of((CHUNK - 1 - j) * B, B)
        gif = gif_ref[pl.ds(rf, B), :]
        gib = gib_ref[pl.ds(rb, B), :]
        ghf = jnp.dot(hf.astype(jnp.bfloat16), whf_ref[...],
                      preferred_element_type=jnp.float32)
        ghb = jnp.dot(hb.astype(jnp.bfloat16), whb_ref[...],
                      preferred_element_type=jnp.float32)

        tf = base_t + j
        rzf = jax.nn.sigmoid(gif[:, :2 * D] + ghf[:, :2 * D])
        nf = jnp.tanh(gif[:, 2 * D:]
                      + rzf[:, :D] * (ghf[:, 2 * D:] + bhnf_ref[...]))
        hf_new = (1.0 - rzf[:, D:]) * nf + rzf[:, D:] * hf
        hf = jnp.where(lens > tf, hf_new, hf)

        tb = (MAX_LEN - 1) - tf
        rzb = jax.nn.sigmoid(gib[:, :2 * D] + ghb[:, :2 * D])
        nb = jnp.tanh(gib[:, 2 * D:]
                      + rzb[:, :D] * (ghb[:, 2 * D:] + bhnb_ref[...]))
        hb_new = (1.0 - rzb[:, D:]) * nb + rzb[:, D:] * hb
        hb = jnp.where(lens > tb, hb_new, hb)
        return hf, hb

    hf_fin, hb_fin = lax.fori_loop(0, CHUNK, step,
                                   (hf_ref[...], hb_ref[...]), unroll=8)
    hf_ref[...] = hf_fin
    hb_ref[...] = hb_fin

    @pl.when(k == NCHUNKS - 1)
    def _head():
        h = jnp.concatenate([hf_ref[...], hb_ref[...]], axis=1)
        u1 = jnp.maximum(
            jnp.dot(h, w1_ref[...], preferred_element_type=jnp.float32), 0.0)
        u2 = jnp.maximum(
            jnp.dot(u1, w2_ref[...], preferred_element_type=jnp.float32), 0.0)
        outs = jnp.dot(u2, w3s_ref[...], preferred_element_type=jnp.float32)
        oute = jnp.dot(u2, w3e_ref[...], preferred_element_type=jnp.float32)
        outt = jnp.dot(u2, w3t_ref[...], preferred_element_type=jnp.float32)

        def soft_off(o, w):
            m = jnp.max(o, axis=1, keepdims=True)
            e = jnp.exp(o - m)
            return (jnp.sum(e * w, axis=1, keepdims=True)
                    / jnp.sum(e, axis=1, keepdims=True))

        so = soft_off(outs, sw_ref[...])
        eo = soft_off(oute, ew_ref[...])
        out_ref[...] = jnp.concatenate([so, eo, outt], axis=1)


def _tc_gru(padded, lens, Wi_f, Wh_f, bc_f, bhn_f, Wi_b, Wh_b, bc_b, bhn_b,
            W1, W2, W3s, W3e, W3t, sw, ew):
    const = lambda shape: pl.BlockSpec(shape, lambda k: (0, 0))
    return pl.pallas_call(
        _tc_body,
        grid=(NCHUNKS,),
        in_specs=[
            pl.BlockSpec((CHUNK * B, D), lambda k: (k, 0)),
            pl.BlockSpec((CHUNK * B, D), lambda k: (NCHUNKS - 1 - k, 0)),
            const((B, 1)),
            const((D, 3 * D)), const((D, 3 * D)),
            const((1, 3 * D)), const((1, D)),
            const((D, 3 * D)), const((D, 3 * D)),
            const((1, 3 * D)), const((1, D)),
            const((2 * D, 256)), const((256, 256)),
            const((256, BINS)), const((256, BINS)), const((256, 5)),
            const((1, BINS)), const((1, BINS)),
        ],
        out_specs=pl.BlockSpec((B, 7), lambda k: (0, 0)),
        out_shape=jax.ShapeDtypeStruct((B, 7), jnp.float32),
        scratch_shapes=[
            pltpu.VMEM((CHUNK * B, 3 * D), jnp.float32),
            pltpu.VMEM((CHUNK * B, 3 * D), jnp.float32),
            pltpu.VMEM((B, D), jnp.float32),
            pltpu.VMEM((B, D), jnp.float32),
        ],
    )(padded, padded, lens, Wi_f, Wh_f, bc_f, bhn_f, Wi_b, Wh_b, bc_b,
      bhn_b, W1, W2, W3s, W3e, W3t, sw, ew)


def kernel(flat, cu_seqlens, Wi_f, Wh_f, bi_f, bh_f, Wi_b, Wh_b, bi_b, bh_b,
           W1, W2, W3, start_w, end_w):
    cu = cu_seqlens.astype(jnp.int32)
    tok = jnp.arange(T, dtype=jnp.int32)
    seg = jnp.searchsorted(cu, tok, side="right").astype(jnp.int32) - 1
    pos = tok - cu[seg]
    dest = (pos * B + seg).reshape(NWORKERS, 2, IDX_W)
    padded = _sc_scatter(flat, dest)

    lens = (cu[1:] - cu[:-1]).reshape(B, 1)
    zero_n = jnp.zeros((D,), jnp.float32)
    bc_f = (bi_f + jnp.concatenate([bh_f[:2 * D], zero_n])).reshape(1, 3 * D)
    bc_b = (bi_b + jnp.concatenate([bh_b[:2 * D], zero_n])).reshape(1, 3 * D)
    bhn_f = bh_f[2 * D:].reshape(1, D)
    bhn_b = bh_b[2 * D:].reshape(1, D)
    W3s = W3[:, :BINS]
    W3e = W3[:, BINS:2 * BINS]
    W3t = W3[:, 2 * BINS:]
    sw = start_w.reshape(1, BINS)
    ew = end_w.reshape(1, BINS)

    bf16 = jnp.bfloat16
    return _tc_gru(padded, lens, Wi_f.astype(bf16), Wh_f.astype(bf16),
                   bc_f, bhn_f, Wi_b.astype(bf16), Wh_b.astype(bf16),
                   bc_b, bhn_b, W1, W2, W3s, W3e, W3t, sw, ew)


# speedup vs baseline: 12.9316x; 1.0099x over previous
import functools

import jax
import jax.numpy as jnp
from jax import lax
from jax.experimental import pallas as pl
from jax.experimental.pallas import tpu as pltpu
from jax.experimental.pallas import tpu_sc as plsc

B = 8
T = 8192
D = 256
MAX_LEN = 2048
BINS = 80
CHUNK = 256
NCHUNKS = MAX_LEN // CHUNK
NWORKERS = 32
ROWS_PER_W = T // NWORKERS
IDX_W = 128


def _sc_scatter(flat, idx):
    mesh = plsc.VectorSubcoreMesh(core_axis_name="c", subcore_axis_name="s")

    @functools.partial(
        pl.kernel,
        out_type=jax.ShapeDtypeStruct((MAX_LEN * B, D), jnp.float32),
        mesh=mesh,
        scratch_types=[
            pltpu.VMEM((2, IDX_W), jnp.int32),
            pltpu.VMEM((ROWS_PER_W, D), jnp.float32),
            pltpu.SemaphoreType.DMA,
            pltpu.SemaphoreType.DMA,
            pltpu.SemaphoreType.DMA,
        ],
    )
    def scatter_kernel(flat_hbm, idx_hbm, out_hbm, idx_v, rows_v,
                       sem_ld0, sem_ld1, sem_st):
        wid = lax.axis_index("s") * 2 + lax.axis_index("c")
        base = wid * ROWS_PER_W
        ld0 = pltpu.async_copy(flat_hbm.at[pl.ds(base, IDX_W)],
                               rows_v.at[pl.ds(0, IDX_W)], sem_ld0)
        ld1 = pltpu.async_copy(flat_hbm.at[pl.ds(base + IDX_W, IDX_W)],
                               rows_v.at[pl.ds(IDX_W, IDX_W)], sem_ld1)
        pltpu.sync_copy(idx_hbm.at[wid], idx_v)
        ld0.wait()
        st0 = pltpu.async_copy(rows_v.at[pl.ds(0, IDX_W)],
                               out_hbm.at[idx_v.at[0]], sem_st)
        ld1.wait()
        st1 = pltpu.async_copy(rows_v.at[pl.ds(IDX_W, IDX_W)],
                               out_hbm.at[idx_v.at[1]], sem_st)
        st0.wait()
        st1.wait()

    return scatter_kernel(flat, idx)


def _tc_body(xf_ref, xb_ref, lens_ref, wif_ref, whf_ref, bcf_ref, bhnf_ref,
             wib_ref, whb_ref, bcb_ref, bhnb_ref, w1_ref, w2_ref, w3s_ref,
             w3e_ref, w3t_ref, sw_ref, ew_ref, out_ref,
             gif_ref, gib_ref, hf_ref, hb_ref):
    k = pl.program_id(0)

    @pl.when(k == 0)
    def _init():
        hf_ref[...] = jnp.zeros((B, D), jnp.float32)
        hb_ref[...] = jnp.zeros((B, D), jnp.float32)

    gif_ref[...] = (
        jnp.dot(xf_ref[...].astype(jnp.bfloat16), wif_ref[...],
                preferred_element_type=jnp.float32)
        + bcf_ref[...])
    gib_ref[...] = (
        jnp.dot(xb_ref[...].astype(jnp.bfloat16), wib_ref[...],
                preferred_element_type=jnp.float32)
        + bcb_ref[...])

    lens = lens_ref[...]
    base_t = k * CHUNK

    def step(j, carry):
        hf = hf_ref[...]
        hb = hb_ref[...]
        rf = pl.multiple_of(j * B, B)
        rb = pl.multiple_of((CHUNK - 1 - j) * B, B)
        gif = gif_ref[pl.ds(rf, B), :]
        gib = gib_ref[pl.ds(rb, B), :]
        ghf = jnp.dot(hf.astype(jnp.bfloat16), whf_ref[...],
                      preferred_element_type=jnp.float32)
        ghb = jnp.dot(hb.astype(jnp.bfloat16), whb_ref[...],
                      preferred_element_type=jnp.float32)

        tf = base_t + j
        rzf = jax.nn.sigmoid(gif[:, :2 * D] + ghf[:, :2 * D])
        nf = jnp.tanh(gif[:, 2 * D:]
                      + rzf[:, :D] * (ghf[:, 2 * D:] + bhnf_ref[...]))
        hf_new = (1.0 - rzf[:, D:]) * nf + rzf[:, D:] * hf
        hf_ref[...] = jnp.where(lens > tf, hf_new, hf)

        tb = (MAX_LEN - 1) - tf
        rzb = jax.nn.sigmoid(gib[:, :2 * D] + ghb[:, :2 * D])
        nb = jnp.tanh(gib[:, 2 * D:]
                      + rzb[:, :D] * (ghb[:, 2 * D:] + bhnb_ref[...]))
        hb_new = (1.0 - rzb[:, D:]) * nb + rzb[:, D:] * hb
        hb_ref[...] = jnp.where(lens > tb, hb_new, hb)
        return carry

    lax.fori_loop(0, CHUNK, step, 0, unroll=8)

    @pl.when(k == NCHUNKS - 1)
    def _head():
        h = jnp.concatenate([hf_ref[...], hb_ref[...]], axis=1)
        u1 = jnp.maximum(
            jnp.dot(h, w1_ref[...], preferred_element_type=jnp.float32), 0.0)
        u2 = jnp.maximum(
            jnp.dot(u1, w2_ref[...], preferred_element_type=jnp.float32), 0.0)
        outs = jnp.dot(u2, w3s_ref[...], preferred_element_type=jnp.float32)
        oute = jnp.dot(u2, w3e_ref[...], preferred_element_type=jnp.float32)
        outt = jnp.dot(u2, w3t_ref[...], preferred_element_type=jnp.float32)

        def soft_off(o, w):
            m = jnp.max(o, axis=1, keepdims=True)
            e = jnp.exp(o - m)
            return (jnp.sum(e * w, axis=1, keepdims=True)
                    / jnp.sum(e, axis=1, keepdims=True))

        so = soft_off(outs, sw_ref[...])
        eo = soft_off(oute, ew_ref[...])
        out_ref[...] = jnp.concatenate([so, eo, outt], axis=1)


def _tc_gru(padded, lens, Wi_f, Wh_f, bc_f, bhn_f, Wi_b, Wh_b, bc_b, bhn_b,
            W1, W2, W3s, W3e, W3t, sw, ew):
    const = lambda shape: pl.BlockSpec(shape, lambda k: (0, 0))
    return pl.pallas_call(
        _tc_body,
        grid=(NCHUNKS,),
        in_specs=[
            pl.BlockSpec((CHUNK * B, D), lambda k: (k, 0)),
            pl.BlockSpec((CHUNK * B, D), lambda k: (NCHUNKS - 1 - k, 0)),
            const((B, 1)),
            const((D, 3 * D)), const((D, 3 * D)),
            const((1, 3 * D)), const((1, D)),
            const((D, 3 * D)), const((D, 3 * D)),
            const((1, 3 * D)), const((1, D)),
            const((2 * D, 256)), const((256, 256)),
            const((256, BINS)), const((256, BINS)), const((256, 5)),
            const((1, BINS)), const((1, BINS)),
        ],
        out_specs=pl.BlockSpec((B, 7), lambda k: (0, 0)),
        out_shape=jax.ShapeDtypeStruct((B, 7), jnp.float32),
        scratch_shapes=[
            pltpu.VMEM((CHUNK * B, 3 * D), jnp.float32),
            pltpu.VMEM((CHUNK * B, 3 * D), jnp.float32),
            pltpu.VMEM((B, D), jnp.float32),
            pltpu.VMEM((B, D), jnp.float32),
        ],
    )(padded, padded, lens, Wi_f, Wh_f, bc_f, bhn_f, Wi_b, Wh_b, bc_b,
      bhn_b, W1, W2, W3s, W3e, W3t, sw, ew)


def kernel(flat, cu_seqlens, Wi_f, Wh_f, bi_f, bh_f, Wi_b, Wh_b, bi_b, bh_b,
           W1, W2, W3, start_w, end_w):
    cu = cu_seqlens.astype(jnp.int32)
    tok = jnp.arange(T, dtype=jnp.int32)
    seg = jnp.searchsorted(cu, tok, side="right").astype(jnp.int32) - 1
    pos = tok - cu[seg]
    dest = (pos * B + seg).reshape(NWORKERS, 2, IDX_W)
    padded = _sc_scatter(flat, dest)

    lens = (cu[1:] - cu[:-1]).reshape(B, 1)
    zero_n = jnp.zeros((D,), jnp.float32)
    bc_f = (bi_f + jnp.concatenate([bh_f[:2 * D], zero_n])).reshape(1, 3 * D)
    bc_b = (bi_b + jnp.concatenate([bh_b[:2 * D], zero_n])).reshape(1, 3 * D)
    bhn_f = bh_f[2 * D:].reshape(1, D)
    bhn_b = bh_b[2 * D:].reshape(1, D)
    W3s = W3[:, :BINS]
    W3e = W3[:, BINS:2 * BINS]
    W3t = W3[:, 2 * BINS:]
    sw = start_w.reshape(1, BINS)
    ew = end_w.reshape(1, BINS)

    bf16 = jnp.bfloat16
    return _tc_gru(padded, lens, Wi_f.astype(bf16), Wh_f.astype(bf16),
                   bc_f, bhn_f, Wi_b.astype(bf16), Wh_b.astype(bf16),
                   bc_b, bhn_b, W1, W2, W3s, W3e, W3t, sw, ew)


# speedup vs baseline: 13.2120x; 1.0217x over previous
import functools

import jax
import jax.numpy as jnp
from jax import lax
from jax.experimental import pallas as pl
from jax.experimental.pallas import tpu as pltpu
from jax.experimental.pallas import tpu_sc as plsc

B = 8
T = 8192
D = 256
MAX_LEN = 2048
BINS = 80
CHUNK = 256
NCHUNKS = MAX_LEN // CHUNK
NWORKERS = 32
ROWS_PER_W = T // NWORKERS
IDX_W = 128


def _sc_scatter(flat, idx):
    mesh = plsc.VectorSubcoreMesh(core_axis_name="c", subcore_axis_name="s")

    @functools.partial(
        pl.kernel,
        out_type=jax.ShapeDtypeStruct((MAX_LEN * B, D), jnp.float32),
        mesh=mesh,
        scratch_types=[
            pltpu.VMEM((2, IDX_W), jnp.int32),
            pltpu.VMEM((ROWS_PER_W, D), jnp.float32),
            pltpu.SemaphoreType.DMA,
            pltpu.SemaphoreType.DMA,
            pltpu.SemaphoreType.DMA,
        ],
    )
    def scatter_kernel(flat_hbm, idx_hbm, out_hbm, idx_v, rows_v,
                       sem_ld0, sem_ld1, sem_st):
        wid = lax.axis_index("s") * 2 + lax.axis_index("c")
        base = wid * ROWS_PER_W
        ld0 = pltpu.async_copy(flat_hbm.at[pl.ds(base, IDX_W)],
                               rows_v.at[pl.ds(0, IDX_W)], sem_ld0)
        ld1 = pltpu.async_copy(flat_hbm.at[pl.ds(base + IDX_W, IDX_W)],
                               rows_v.at[pl.ds(IDX_W, IDX_W)], sem_ld1)
        pltpu.sync_copy(idx_hbm.at[wid], idx_v)
        ld0.wait()
        st0 = pltpu.async_copy(rows_v.at[pl.ds(0, IDX_W)],
                               out_hbm.at[idx_v.at[0]], sem_st)
        ld1.wait()
        st1 = pltpu.async_copy(rows_v.at[pl.ds(IDX_W, IDX_W)],
                               out_hbm.at[idx_v.at[1]], sem_st)
        st0.wait()
        st1.wait()

    return scatter_kernel(flat, idx)


def _tc_body(xf_ref, xb_ref, lens_ref, wif_ref, whf_ref, bcf_ref, bhnf_ref,
             wib_ref, whb_ref, bcb_ref, bhnb_ref, w1_ref, w2_ref, w3s_ref,
             w3e_ref, w3t_ref, sw_ref, ew_ref, out_ref,
             gif_ref, gib_ref, hf_ref, hb_ref):
    k = pl.program_id(0)

    @pl.when(k == 0)
    def _init():
        hf_ref[...] = jnp.zeros((B, D), jnp.float32)
        hb_ref[...] = jnp.zeros((B, D), jnp.float32)

    gif_ref[...] = (
        jnp.dot(xf_ref[...].astype(jnp.bfloat16), wif_ref[...],
                preferred_element_type=jnp.float32)
        + bcf_ref[...])
    gib_ref[...] = (
        jnp.dot(xb_ref[...].astype(jnp.bfloat16), wib_ref[...],
                preferred_element_type=jnp.float32)
        + bcb_ref[...])

    lens = lens_ref[...]
    base_t = k * CHUNK

    def step(j, carry):
        hf = hf_ref[...]
        hb = hb_ref[...]
        rf = pl.multiple_of(j * B, B)
        rb = pl.multiple_of((CHUNK - 1 - j) * B, B)
        gif = gif_ref[pl.ds(rf, B), :]
        gib = gib_ref[pl.ds(rb, B), :]
        ghf = jnp.dot(hf.astype(jnp.bfloat16), whf_ref[...],
                      preferred_element_type=jnp.float32)
        ghb = jnp.dot(hb.astype(jnp.bfloat16), whb_ref[...],
                      preferred_element_type=jnp.float32)

        tf = base_t + j
        rzf = jax.nn.sigmoid(gif[:, :2 * D] + ghf[:, :2 * D])
        nf = jnp.tanh(gif[:, 2 * D:]
                      + rzf[:, :D] * (ghf[:, 2 * D:] + bhnf_ref[...]))
        hf_new = (1.0 - rzf[:, D:]) * nf + rzf[:, D:] * hf
        hf_ref[...] = jnp.where(lens > tf, hf_new, hf)

        tb = (MAX_LEN - 1) - tf
        rzb = jax.nn.sigmoid(gib[:, :2 * D] + ghb[:, :2 * D])
        nb = jnp.tanh(gib[:, 2 * D:]
                      + rzb[:, :D] * (ghb[:, 2 * D:] + bhnb_ref[...]))
        hb_new = (1.0 - rzb[:, D:]) * nb + rzb[:, D:] * hb
        hb_ref[...] = jnp.where(lens > tb, hb_new, hb)
        return carry

    lax.fori_loop(0, CHUNK, step, 0, unroll=16)

    @pl.when(k == NCHUNKS - 1)
    def _head():
        h = jnp.concatenate([hf_ref[...], hb_ref[...]], axis=1)
        u1 = jnp.maximum(
            jnp.dot(h, w1_ref[...], preferred_element_type=jnp.float32), 0.0)
        u2 = jnp.maximum(
            jnp.dot(u1, w2_ref[...], preferred_element_type=jnp.float32), 0.0)
        outs = jnp.dot(u2, w3s_ref[...], preferred_element_type=jnp.float32)
        oute = jnp.dot(u2, w3e_ref[...], preferred_element_type=jnp.float32)
        outt = jnp.dot(u2, w3t_ref[...], preferred_element_type=jnp.float32)

        def soft_off(o, w):
            m = jnp.max(o, axis=1, keepdims=True)
            e = jnp.exp(o - m)
            return (jnp.sum(e * w, axis=1, keepdims=True)
                    / jnp.sum(e, axis=1, keepdims=True))

        so = soft_off(outs, sw_ref[...])
        eo = soft_off(oute, ew_ref[...])
        out_ref[...] = jnp.concatenate([so, eo, outt], axis=1)


def _tc_gru(padded, lens, Wi_f, Wh_f, bc_f, bhn_f, Wi_b, Wh_b, bc_b, bhn_b,
            W1, W2, W3s, W3e, W3t, sw, ew):
    const = lambda shape: pl.BlockSpec(shape, lambda k: (0, 0))
    return pl.pallas_call(
        _tc_body,
        grid=(NCHUNKS,),
        in_specs=[
            pl.BlockSpec((CHUNK * B, D), lambda k: (k, 0)),
            pl.BlockSpec((CHUNK * B, D), lambda k: (NCHUNKS - 1 - k, 0)),
            const((B, 1)),
            const((D, 3 * D)), const((D, 3 * D)),
            const((1, 3 * D)), const((1, D)),
            const((D, 3 * D)), const((D, 3 * D)),
            const((1, 3 * D)), const((1, D)),
            const((2 * D, 256)), const((256, 256)),
            const((256, BINS)), const((256, BINS)), const((256, 5)),
            const((1, BINS)), const((1, BINS)),
        ],
        out_specs=pl.BlockSpec((B, 7), lambda k: (0, 0)),
        out_shape=jax.ShapeDtypeStruct((B, 7), jnp.float32),
        scratch_shapes=[
            pltpu.VMEM((CHUNK * B, 3 * D), jnp.float32),
            pltpu.VMEM((CHUNK * B, 3 * D), jnp.float32),
            pltpu.VMEM((B, D), jnp.float32),
            pltpu.VMEM((B, D), jnp.float32),
        ],
    )(padded, padded, lens, Wi_f, Wh_f, bc_f, bhn_f, Wi_b, Wh_b, bc_b,
      bhn_b, W1, W2, W3s, W3e, W3t, sw, ew)


def kernel(flat, cu_seqlens, Wi_f, Wh_f, bi_f, bh_f, Wi_b, Wh_b, bi_b, bh_b,
           W1, W2, W3, start_w, end_w):
    cu = cu_seqlens.astype(jnp.int32)
    tok = jnp.arange(T, dtype=jnp.int32)
    seg = jnp.searchsorted(cu, tok, side="right").astype(jnp.int32) - 1
    pos = tok - cu[seg]
    dest = (pos * B + seg).reshape(NWORKERS, 2, IDX_W)
    padded = _sc_scatter(flat, dest)

    lens = (cu[1:] - cu[:-1]).reshape(B, 1)
    zero_n = jnp.zeros((D,), jnp.float32)
    bc_f = (bi_f + jnp.concatenate([bh_f[:2 * D], zero_n])).reshape(1, 3 * D)
    bc_b = (bi_b + jnp.concatenate([bh_b[:2 * D], zero_n])).reshape(1, 3 * D)
    bhn_f = bh_f[2 * D:].reshape(1, D)
    bhn_b = bh_b[2 * D:].reshape(1, D)
    W3s = W3[:, :BINS]
    W3e = W3[:, BINS:2 * BINS]
    W3t = W3[:, 2 * BINS:]
    sw = start_w.reshape(1, BINS)
    ew = end_w.reshape(1, BINS)

    bf16 = jnp.bfloat16
    return _tc_gru(padded, lens, Wi_f.astype(bf16), Wh_f.astype(bf16),
                   bc_f, bhn_f, Wi_b.astype(bf16), Wh_b.astype(bf16),
                   bc_b, bhn_b, W1, W2, W3s, W3e, W3t, sw, ew)


# speedup vs baseline: 13.3726x; 1.0122x over previous
import functools

import jax
import jax.numpy as jnp
from jax import lax
from jax.experimental import pallas as pl
from jax.experimental.pallas import tpu as pltpu
from jax.experimental.pallas import tpu_sc as plsc

B = 8
T = 8192
D = 256
MAX_LEN = 2048
BINS = 80
CHUNK = 256
NCHUNKS = MAX_LEN // CHUNK
NWORKERS = 32
ROWS_PER_W = T // NWORKERS
IDX_W = 128


def _sc_scatter(flat, idx):
    mesh = plsc.VectorSubcoreMesh(core_axis_name="c", subcore_axis_name="s")

    @functools.partial(
        pl.kernel,
        out_type=jax.ShapeDtypeStruct((MAX_LEN * B, D), jnp.float32),
        mesh=mesh,
        scratch_types=[
            pltpu.VMEM((2, IDX_W), jnp.int32),
            pltpu.VMEM((ROWS_PER_W, D), jnp.float32),
            pltpu.SemaphoreType.DMA,
            pltpu.SemaphoreType.DMA,
            pltpu.SemaphoreType.DMA,
        ],
    )
    def scatter_kernel(flat_hbm, idx_hbm, out_hbm, idx_v, rows_v,
                       sem_ld0, sem_ld1, sem_st):
        wid = lax.axis_index("s") * 2 + lax.axis_index("c")
        base = wid * ROWS_PER_W
        ld0 = pltpu.async_copy(flat_hbm.at[pl.ds(base, IDX_W)],
                               rows_v.at[pl.ds(0, IDX_W)], sem_ld0)
        ld1 = pltpu.async_copy(flat_hbm.at[pl.ds(base + IDX_W, IDX_W)],
                               rows_v.at[pl.ds(IDX_W, IDX_W)], sem_ld1)
        pltpu.sync_copy(idx_hbm.at[wid], idx_v)
        ld0.wait()
        st0 = pltpu.async_copy(rows_v.at[pl.ds(0, IDX_W)],
                               out_hbm.at[idx_v.at[0]], sem_st)
        ld1.wait()
        st1 = pltpu.async_copy(rows_v.at[pl.ds(IDX_W, IDX_W)],
                               out_hbm.at[idx_v.at[1]], sem_st)
        st0.wait()
        st1.wait()

    return scatter_kernel(flat, idx)


def _tc_body(xf_ref, xb_ref, lens_ref, wif_ref, whf_ref, bcf_ref, bhnf_ref,
             wib_ref, whb_ref, bcb_ref, bhnb_ref, w1_ref, w2_ref, w3s_ref,
             w3e_ref, w3t_ref, sw_ref, ew_ref, out_ref,
             gif_ref, gib_ref, hf_ref, hb_ref):
    k = pl.program_id(0)

    @pl.when(k == 0)
    def _init():
        hf_ref[...] = jnp.zeros((B, D), jnp.float32)
        hb_ref[...] = jnp.zeros((B, D), jnp.float32)

    gif_ref[...] = (
        jnp.dot(xf_ref[...].astype(jnp.bfloat16), wif_ref[...],
                preferred_element_type=jnp.float32)
        + bcf_ref[...])
    gib_ref[...] = (
        jnp.dot(xb_ref[...].astype(jnp.bfloat16), wib_ref[...],
                preferred_element_type=jnp.float32)
        + bcb_ref[...])

    lens = lens_ref[...]
    base_t = k * CHUNK

    def step(j, carry):
        hf = hf_ref[...]
        hb = hb_ref[...]
        rf = pl.multiple_of(j * B, B)
        rb = pl.multiple_of((CHUNK - 1 - j) * B, B)
        gif = gif_ref[pl.ds(rf, B), :]
        gib = gib_ref[pl.ds(rb, B), :]
        ghf = jnp.dot(hf.astype(jnp.bfloat16), whf_ref[...],
                      preferred_element_type=jnp.float32)
        ghb = jnp.dot(hb.astype(jnp.bfloat16), whb_ref[...],
                      preferred_element_type=jnp.float32)

        tf = base_t + j
        rzf = jax.nn.sigmoid(gif[:, :2 * D] + ghf[:, :2 * D])
        nf = jnp.tanh(gif[:, 2 * D:]
                      + rzf[:, :D] * (ghf[:, 2 * D:] + bhnf_ref[...]))
        hf_new = (1.0 - rzf[:, D:]) * nf + rzf[:, D:] * hf
        hf_ref[...] = jnp.where(lens > tf, hf_new, hf)

        tb = (MAX_LEN - 1) - tf
        rzb = jax.nn.sigmoid(gib[:, :2 * D] + ghb[:, :2 * D])
        nb = jnp.tanh(gib[:, 2 * D:]
                      + rzb[:, :D] * (ghb[:, 2 * D:] + bhnb_ref[...]))
        hb_new = (1.0 - rzb[:, D:]) * nb + rzb[:, D:] * hb
        hb_ref[...] = jnp.where(lens > tb, hb_new, hb)
        return carry

    lax.fori_loop(0, CHUNK, step, 0, unroll=32)

    @pl.when(k == NCHUNKS - 1)
    def _head():
        h = jnp.concatenate([hf_ref[...], hb_ref[...]], axis=1)
        u1 = jnp.maximum(
            jnp.dot(h, w1_ref[...], preferred_element_type=jnp.float32), 0.0)
        u2 = jnp.maximum(
            jnp.dot(u1, w2_ref[...], preferred_element_type=jnp.float32), 0.0)
        outs = jnp.dot(u2, w3s_ref[...], preferred_element_type=jnp.float32)
        oute = jnp.dot(u2, w3e_ref[...], preferred_element_type=jnp.float32)
        outt = jnp.dot(u2, w3t_ref[...], preferred_element_type=jnp.float32)

        def soft_off(o, w):
            m = jnp.max(o, axis=1, keepdims=True)
            e = jnp.exp(o - m)
            return (jnp.sum(e * w, axis=1, keepdims=True)
                    / jnp.sum(e, axis=1, keepdims=True))

        so = soft_off(outs, sw_ref[...])
        eo = soft_off(oute, ew_ref[...])
        out_ref[...] = jnp.concatenate([so, eo, outt], axis=1)


def _tc_gru(padded, lens, Wi_f, Wh_f, bc_f, bhn_f, Wi_b, Wh_b, bc_b, bhn_b,
            W1, W2, W3s, W3e, W3t, sw, ew):
    const = lambda shape: pl.BlockSpec(shape, lambda k: (0, 0))
    return pl.pallas_call(
        _tc_body,
        grid=(NCHUNKS,),
        in_specs=[
            pl.BlockSpec((CHUNK * B, D), lambda k: (k, 0)),
            pl.BlockSpec((CHUNK * B, D), lambda k: (NCHUNKS - 1 - k, 0)),
            const((B, 1)),
            const((D, 3 * D)), const((D, 3 * D)),
            const((1, 3 * D)), const((1, D)),
            const((D, 3 * D)), const((D, 3 * D)),
            const((1, 3 * D)), const((1, D)),
            const((2 * D, 256)), const((256, 256)),
            const((256, BINS)), const((256, BINS)), const((256, 5)),
            const((1, BINS)), const((1, BINS)),
        ],
        out_specs=pl.BlockSpec((B, 7), lambda k: (0, 0)),
        out_shape=jax.ShapeDtypeStruct((B, 7), jnp.float32),
        scratch_shapes=[
            pltpu.VMEM((CHUNK * B, 3 * D), jnp.float32),
            pltpu.VMEM((CHUNK * B, 3 * D), jnp.float32),
            pltpu.VMEM((B, D), jnp.float32),
            pltpu.VMEM((B, D), jnp.float32),
        ],
    )(padded, padded, lens, Wi_f, Wh_f, bc_f, bhn_f, Wi_b, Wh_b, bc_b,
      bhn_b, W1, W2, W3s, W3e, W3t, sw, ew)


def kernel(flat, cu_seqlens, Wi_f, Wh_f, bi_f, bh_f, Wi_b, Wh_b, bi_b, bh_b,
           W1, W2, W3, start_w, end_w):
    cu = cu_seqlens.astype(jnp.int32)
    tok = jnp.arange(T, dtype=jnp.int32)
    seg = jnp.searchsorted(cu, tok, side="right").astype(jnp.int32) - 1
    pos = tok - cu[seg]
    dest = (pos * B + seg).reshape(NWORKERS, 2, IDX_W)
    padded = _sc_scatter(flat, dest)

    lens = (cu[1:] - cu[:-1]).reshape(B, 1)
    zero_n = jnp.zeros((D,), jnp.float32)
    bc_f = (bi_f + jnp.concatenate([bh_f[:2 * D], zero_n])).reshape(1, 3 * D)
    bc_b = (bi_b + jnp.concatenate([bh_b[:2 * D], zero_n])).reshape(1, 3 * D)
    bhn_f = bh_f[2 * D:].reshape(1, D)
    bhn_b = bh_b[2 * D:].reshape(1, D)
    W3s = W3[:, :BINS]
    W3e = W3[:, BINS:2 * BINS]
    W3t = W3[:, 2 * BINS:]
    sw = start_w.reshape(1, BINS)
    ew = end_w.reshape(1, BINS)

    bf16 = jnp.bfloat16
    return _tc_gru(padded, lens, Wi_f.astype(bf16), Wh_f.astype(bf16),
                   bc_f, bhn_f, Wi_b.astype(bf16), Wh_b.astype(bf16),
                   bc_b, bhn_b, W1, W2, W3s, W3e, W3t, sw, ew)


# speedup vs baseline: 13.4386x; 1.0049x over previous
import functools

import jax
import jax.numpy as jnp
from jax import lax
from jax.experimental import pallas as pl
from jax.experimental.pallas import tpu as pltpu
from jax.experimental.pallas import tpu_sc as plsc

B = 8
T = 8192
D = 256
MAX_LEN = 2048
BINS = 80
CHUNK = 256
NCHUNKS = MAX_LEN // CHUNK
NWORKERS = 32
ROWS_PER_W = T // NWORKERS
IDX_W = 128


def _sc_scatter(flat, idx):
    mesh = plsc.VectorSubcoreMesh(core_axis_name="c", subcore_axis_name="s")

    @functools.partial(
        pl.kernel,
        out_type=jax.ShapeDtypeStruct((MAX_LEN * B, D), jnp.float32),
        mesh=mesh,
        scratch_types=[
            pltpu.VMEM((2, IDX_W), jnp.int32),
            pltpu.VMEM((ROWS_PER_W, D), jnp.float32),
            pltpu.SemaphoreType.DMA,
            pltpu.SemaphoreType.DMA,
            pltpu.SemaphoreType.DMA,
        ],
    )
    def scatter_kernel(flat_hbm, idx_hbm, out_hbm, idx_v, rows_v,
                       sem_ld0, sem_ld1, sem_st):
        wid = lax.axis_index("s") * 2 + lax.axis_index("c")
        base = wid * ROWS_PER_W
        ld0 = pltpu.async_copy(flat_hbm.at[pl.ds(base, IDX_W)],
                               rows_v.at[pl.ds(0, IDX_W)], sem_ld0)
        ld1 = pltpu.async_copy(flat_hbm.at[pl.ds(base + IDX_W, IDX_W)],
                               rows_v.at[pl.ds(IDX_W, IDX_W)], sem_ld1)
        pltpu.sync_copy(idx_hbm.at[wid], idx_v)
        ld0.wait()
        st0 = pltpu.async_copy(rows_v.at[pl.ds(0, IDX_W)],
                               out_hbm.at[idx_v.at[0]], sem_st)
        ld1.wait()
        st1 = pltpu.async_copy(rows_v.at[pl.ds(IDX_W, IDX_W)],
                               out_hbm.at[idx_v.at[1]], sem_st)
        st0.wait()
        st1.wait()

    return scatter_kernel(flat, idx)


def _tc_body(xf_ref, xb_ref, lens_ref, wif_ref, whf_ref, bcf_ref, bhnf_ref,
             wib_ref, whb_ref, bcb_ref, bhnb_ref, w1_ref, w2_ref, w3s_ref,
             w3e_ref, w3t_ref, sw_ref, ew_ref, out_ref,
             gif_ref, gib_ref, hf_ref, hb_ref):
    k = pl.program_id(0)

    @pl.when(k == 0)
    def _init():
        hf_ref[...] = jnp.zeros((B, D), jnp.float32)
        hb_ref[...] = jnp.zeros((B, D), jnp.float32)

    gif_ref[...] = (
        jnp.dot(xf_ref[...].astype(jnp.bfloat16), wif_ref[...],
                preferred_element_type=jnp.float32)
        + bcf_ref[...])
    gib_ref[...] = (
        jnp.dot(xb_ref[...].astype(jnp.bfloat16), wib_ref[...],
                preferred_element_type=jnp.float32)
        + bcb_ref[...])

    lens = lens_ref[...]
    base_t = k * CHUNK

    def step(j, carry):
        hf = hf_ref[...]
        hb = hb_ref[...]
        rf = pl.multiple_of(j * B, B)
        rb = pl.multiple_of((CHUNK - 1 - j) * B, B)
        gif = gif_ref[pl.ds(rf, B), :]
        gib = gib_ref[pl.ds(rb, B), :]
        ghf = jnp.dot(hf.astype(jnp.bfloat16), whf_ref[...],
                      preferred_element_type=jnp.float32)
        ghb = jnp.dot(hb.astype(jnp.bfloat16), whb_ref[...],
                      preferred_element_type=jnp.float32)

        tf = base_t + j
        rzf = jax.nn.sigmoid(gif[:, :2 * D] + ghf[:, :2 * D])
        nf = jnp.tanh(gif[:, 2 * D:]
                      + rzf[:, :D] * (ghf[:, 2 * D:] + bhnf_ref[...]))
        hf_new = (1.0 - rzf[:, D:]) * nf + rzf[:, D:] * hf
        hf_ref[...] = jnp.where(lens > tf, hf_new, hf)

        tb = (MAX_LEN - 1) - tf
        rzb = jax.nn.sigmoid(gib[:, :2 * D] + ghb[:, :2 * D])
        nb = jnp.tanh(gib[:, 2 * D:]
                      + rzb[:, :D] * (ghb[:, 2 * D:] + bhnb_ref[...]))
        hb_new = (1.0 - rzb[:, D:]) * nb + rzb[:, D:] * hb
        hb_ref[...] = jnp.where(lens > tb, hb_new, hb)
        return carry

    lax.fori_loop(0, CHUNK, step, 0, unroll=64)

    @pl.when(k == NCHUNKS - 1)
    def _head():
        h = jnp.concatenate([hf_ref[...], hb_ref[...]], axis=1)
        u1 = jnp.maximum(
            jnp.dot(h, w1_ref[...], preferred_element_type=jnp.float32), 0.0)
        u2 = jnp.maximum(
            jnp.dot(u1, w2_ref[...], preferred_element_type=jnp.float32), 0.0)
        outs = jnp.dot(u2, w3s_ref[...], preferred_element_type=jnp.float32)
        oute = jnp.dot(u2, w3e_ref[...], preferred_element_type=jnp.float32)
        outt = jnp.dot(u2, w3t_ref[...], preferred_element_type=jnp.float32)

        def soft_off(o, w):
            m = jnp.max(o, axis=1, keepdims=True)
            e = jnp.exp(o - m)
            return (jnp.sum(e * w, axis=1, keepdims=True)
                    / jnp.sum(e, axis=1, keepdims=True))

        so = soft_off(outs, sw_ref[...])
        eo = soft_off(oute, ew_ref[...])
        out_ref[...] = jnp.concatenate([so, eo, outt], axis=1)


def _tc_gru(padded, lens, Wi_f, Wh_f, bc_f, bhn_f, Wi_b, Wh_b, bc_b, bhn_b,
            W1, W2, W3s, W3e, W3t, sw, ew):
    const = lambda shape: pl.BlockSpec(shape, lambda k: (0, 0))
    return pl.pallas_call(
        _tc_body,
        grid=(NCHUNKS,),
        in_specs=[
            pl.BlockSpec((CHUNK * B, D), lambda k: (k, 0)),
            pl.BlockSpec((CHUNK * B, D), lambda k: (NCHUNKS - 1 - k, 0)),
            const((B, 1)),
            const((D, 3 * D)), const((D, 3 * D)),
            const((1, 3 * D)), const((1, D)),
            const((D, 3 * D)), const((D, 3 * D)),
            const((1, 3 * D)), const((1, D)),
            const((2 * D, 256)), const((256, 256)),
            const((256, BINS)), const((256, BINS)), const((256, 5)),
            const((1, BINS)), const((1, BINS)),
        ],
        out_specs=pl.BlockSpec((B, 7), lambda k: (0, 0)),
        out_shape=jax.ShapeDtypeStruct((B, 7), jnp.float32),
        scratch_shapes=[
            pltpu.VMEM((CHUNK * B, 3 * D), jnp.float32),
            pltpu.VMEM((CHUNK * B, 3 * D), jnp.float32),
            pltpu.VMEM((B, D), jnp.float32),
            pltpu.VMEM((B, D), jnp.float32),
        ],
    )(padded, padded, lens, Wi_f, Wh_f, bc_f, bhn_f, Wi_b, Wh_b, bc_b,
      bhn_b, W1, W2, W3s, W3e, W3t, sw, ew)


def kernel(flat, cu_seqlens, Wi_f, Wh_f, bi_f, bh_f, Wi_b, Wh_b, bi_b, bh_b,
           W1, W2, W3, start_w, end_w):
    cu = cu_seqlens.astype(jnp.int32)
    tok = jnp.arange(T, dtype=jnp.int32)
    seg = jnp.searchsorted(cu, tok, side="right").astype(jnp.int32) - 1
    pos = tok - cu[seg]
    dest = (pos * B + seg).reshape(NWORKERS, 2, IDX_W)
    padded = _sc_scatter(flat, dest)

    lens = (cu[1:] - cu[:-1]).reshape(B, 1)
    zero_n = jnp.zeros((D,), jnp.float32)
    bc_f = (bi_f + jnp.concatenate([bh_f[:2 * D], zero_n])).reshape(1, 3 * D)
    bc_b = (bi_b + jnp.concatenate([bh_b[:2 * D], zero_n])).reshape(1, 3 * D)
    bhn_f = bh_f[2 * D:].reshape(1, D)
    bhn_b = bh_b[2 * D:].reshape(1, D)
    W3s = W3[:, :BINS]
    W3e = W3[:, BINS:2 * BINS]
    W3t = W3[:, 2 * BINS:]
    sw = start_w.reshape(1, BINS)
    ew = end_w.reshape(1, BINS)

    bf16 = jnp.bfloat16
    return _tc_gru(padded, lens, Wi_f.astype(bf16), Wh_f.astype(bf16),
                   bc_f, bhn_f, Wi_b.astype(bf16), Wh_b.astype(bf16),
                   bc_b, bhn_b, W1, W2, W3s, W3e, W3t, sw, ew)


# speedup vs baseline: 13.4919x; 1.0040x over previous
import functools

import jax
import jax.numpy as jnp
from jax import lax
from jax.experimental import pallas as pl
from jax.experimental.pallas import tpu as pltpu
from jax.experimental.pallas import tpu_sc as plsc

B = 8
T = 8192
D = 256
MAX_LEN = 2048
BINS = 80
CHUNK = 256
NCHUNKS = MAX_LEN // CHUNK
NWORKERS = 32
ROWS_PER_W = T // NWORKERS
IDX_W = 128


def _sc_scatter(flat, idx):
    mesh = plsc.VectorSubcoreMesh(core_axis_name="c", subcore_axis_name="s")

    @functools.partial(
        pl.kernel,
        out_type=jax.ShapeDtypeStruct((MAX_LEN * B, D), jnp.float32),
        mesh=mesh,
        scratch_types=[
            pltpu.VMEM((2, IDX_W), jnp.int32),
            pltpu.VMEM((ROWS_PER_W, D), jnp.float32),
            pltpu.SemaphoreType.DMA,
            pltpu.SemaphoreType.DMA,
            pltpu.SemaphoreType.DMA,
        ],
    )
    def scatter_kernel(flat_hbm, idx_hbm, out_hbm, idx_v, rows_v,
                       sem_ld0, sem_ld1, sem_st):
        wid = lax.axis_index("s") * 2 + lax.axis_index("c")
        base = wid * ROWS_PER_W
        ld0 = pltpu.async_copy(flat_hbm.at[pl.ds(base, IDX_W)],
                               rows_v.at[pl.ds(0, IDX_W)], sem_ld0)
        ld1 = pltpu.async_copy(flat_hbm.at[pl.ds(base + IDX_W, IDX_W)],
                               rows_v.at[pl.ds(IDX_W, IDX_W)], sem_ld1)
        pltpu.sync_copy(idx_hbm.at[wid], idx_v)
        ld0.wait()
        st0 = pltpu.async_copy(rows_v.at[pl.ds(0, IDX_W)],
                               out_hbm.at[idx_v.at[0]], sem_st)
        ld1.wait()
        st1 = pltpu.async_copy(rows_v.at[pl.ds(IDX_W, IDX_W)],
                               out_hbm.at[idx_v.at[1]], sem_st)
        st0.wait()
        st1.wait()

    return scatter_kernel(flat, idx)


def _tc_body(xf_ref, xb_ref, lens_ref, wif_ref, whf_ref, bcf_ref, bhnf_ref,
             wib_ref, whb_ref, bcb_ref, bhnb_ref, w1_ref, w2_ref, w3s_ref,
             w3e_ref, w3t_ref, sw_ref, ew_ref, out_ref,
             gif_ref, gib_ref, hf_ref, hb_ref):
    k = pl.program_id(0)

    @pl.when(k == 0)
    def _init():
        hf_ref[...] = jnp.zeros((B, D), jnp.float32)
        hb_ref[...] = jnp.zeros((B, D), jnp.float32)

    gif_ref[...] = (
        jnp.dot(xf_ref[...].astype(jnp.bfloat16), wif_ref[...],
                preferred_element_type=jnp.float32)
        + bcf_ref[...])
    gib_ref[...] = (
        jnp.dot(xb_ref[...].astype(jnp.bfloat16), wib_ref[...],
                preferred_element_type=jnp.float32)
        + bcb_ref[...])

    lens = lens_ref[...]
    base_t = k * CHUNK

    def step(j, carry):
        hf = hf_ref[...]
        hb = hb_ref[...]
        rf = pl.multiple_of(j * B, B)
        rb = pl.multiple_of((CHUNK - 1 - j) * B, B)
        gif = gif_ref[pl.ds(rf, B), :]
        gib = gib_ref[pl.ds(rb, B), :]
        ghf = jnp.dot(hf.astype(jnp.bfloat16), whf_ref[...],
                      preferred_element_type=jnp.float32)
        ghb = jnp.dot(hb.astype(jnp.bfloat16), whb_ref[...],
                      preferred_element_type=jnp.float32)

        tf = base_t + j
        rzf = jax.nn.sigmoid(gif[:, :2 * D] + ghf[:, :2 * D])
        nf = jnp.tanh(gif[:, 2 * D:]
                      + rzf[:, :D] * (ghf[:, 2 * D:] + bhnf_ref[...]))
        hf_new = (1.0 - rzf[:, D:]) * nf + rzf[:, D:] * hf
        hf_ref[...] = jnp.where(lens > tf, hf_new, hf)

        tb = (MAX_LEN - 1) - tf
        rzb = jax.nn.sigmoid(gib[:, :2 * D] + ghb[:, :2 * D])
        nb = jnp.tanh(gib[:, 2 * D:]
                      + rzb[:, :D] * (ghb[:, 2 * D:] + bhnb_ref[...]))
        hb_new = (1.0 - rzb[:, D:]) * nb + rzb[:, D:] * hb
        hb_ref[...] = jnp.where(lens > tb, hb_new, hb)
        return carry

    lax.fori_loop(0, CHUNK, step, 0, unroll=128)

    @pl.when(k == NCHUNKS - 1)
    def _head():
        h = jnp.concatenate([hf_ref[...], hb_ref[...]], axis=1)
        u1 = jnp.maximum(
            jnp.dot(h, w1_ref[...], preferred_element_type=jnp.float32), 0.0)
        u2 = jnp.maximum(
            jnp.dot(u1, w2_ref[...], preferred_element_type=jnp.float32), 0.0)
        outs = jnp.dot(u2, w3s_ref[...], preferred_element_type=jnp.float32)
        oute = jnp.dot(u2, w3e_ref[...], preferred_element_type=jnp.float32)
        outt = jnp.dot(u2, w3t_ref[...], preferred_element_type=jnp.float32)

        def soft_off(o, w):
            m = jnp.max(o, axis=1, keepdims=True)
            e = jnp.exp(o - m)
            return (jnp.sum(e * w, axis=1, keepdims=True)
                    / jnp.sum(e, axis=1, keepdims=True))

        so = soft_off(outs, sw_ref[...])
        eo = soft_off(oute, ew_ref[...])
        out_ref[...] = jnp.concatenate([so, eo, outt], axis=1)


def _tc_gru(padded, lens, Wi_f, Wh_f, bc_f, bhn_f, Wi_b, Wh_b, bc_b, bhn_b,
            W1, W2, W3s, W3e, W3t, sw, ew):
    const = lambda shape: pl.BlockSpec(shape, lambda k: (0, 0))
    return pl.pallas_call(
        _tc_body,
        grid=(NCHUNKS,),
        in_specs=[
            pl.BlockSpec((CHUNK * B, D), lambda k: (k, 0)),
            pl.BlockSpec((CHUNK * B, D), lambda k: (NCHUNKS - 1 - k, 0)),
            const((B, 1)),
            const((D, 3 * D)), const((D, 3 * D)),
            const((1, 3 * D)), const((1, D)),
            const((D, 3 * D)), const((D, 3 * D)),
            const((1, 3 * D)), const((1, D)),
            const((2 * D, 256)), const((256, 256)),
            const((256, BINS)), const((256, BINS)), const((256, 5)),
            const((1, BINS)), const((1, BINS)),
        ],
        out_specs=pl.BlockSpec((B, 7), lambda k: (0, 0)),
        out_shape=jax.ShapeDtypeStruct((B, 7), jnp.float32),
        scratch_shapes=[
            pltpu.VMEM((CHUNK * B, 3 * D), jnp.float32),
            pltpu.VMEM((CHUNK * B, 3 * D), jnp.float32),
            pltpu.VMEM((B, D), jnp.float32),
            pltpu.VMEM((B, D), jnp.float32),
        ],
    )(padded, padded, lens, Wi_f, Wh_f, bc_f, bhn_f, Wi_b, Wh_b, bc_b,
      bhn_b, W1, W2, W3s, W3e, W3t, sw, ew)


def kernel(flat, cu_seqlens, Wi_f, Wh_f, bi_f, bh_f, Wi_b, Wh_b, bi_b, bh_b,
           W1, W2, W3, start_w, end_w):
    cu = cu_seqlens.astype(jnp.int32)
    tok = jnp.arange(T, dtype=jnp.int32)
    seg = jnp.searchsorted(cu, tok, side="right").astype(jnp.int32) - 1
    pos = tok - cu[seg]
    dest = (pos * B + seg).reshape(NWORKERS, 2, IDX_W)
    padded = _sc_scatter(flat, dest)

    lens = (cu[1:] - cu[:-1]).reshape(B, 1)
    zero_n = jnp.zeros((D,), jnp.float32)
    bc_f = (bi_f + jnp.concatenate([bh_f[:2 * D], zero_n])).reshape(1, 3 * D)
    bc_b = (bi_b + jnp.concatenate([bh_b[:2 * D], zero_n])).reshape(1, 3 * D)
    bhn_f = bh_f[2 * D:].reshape(1, D)
    bhn_b = bh_b[2 * D:].reshape(1, D)
    W3s = W3[:, :BINS]
    W3e = W3[:, BINS:2 * BINS]
    W3t = W3[:, 2 * BINS:]
    sw = start_w.reshape(1, BINS)
    ew = end_w.reshape(1, BINS)

    bf16 = jnp.bfloat16
    return _tc_gru(padded, lens, Wi_f.astype(bf16), Wh_f.astype(bf16),
                   bc_f, bhn_f, Wi_b.astype(bf16), Wh_b.astype(bf16),
                   bc_b, bhn_b, W1, W2, W3s, W3e, W3t, sw, ew)
